# baseline (speedup 1.0000x reference)
import functools

import jax
import jax.numpy as jnp
from jax.experimental import pallas as pl
from jax.experimental.pallas import tpu as pltpu

_LOW_BOUND = 1
_LOG2E = 1.4426950408889634

_TR = 2048
_SUB = 512


def _mlp_body(x_ref, w1_ref, b1_ref, w2_ref, b2_ref, d_ref):
    for k in range(_TR // _SUB):
        xs = x_ref[k * _SUB:(k + 1) * _SUB, :]
        h = jnp.dot(xs, w1_ref[...], preferred_element_type=jnp.float32)
        h = jnp.maximum(h + b1_ref[...], 0.0)
        logits = jnp.dot(h, w2_ref[...], preferred_element_type=jnp.float32)
        logits = logits + b2_ref[...]
        d_ref[k * _SUB:(k + 1) * _SUB, :] = logits[:, 1:2] - logits[:, 0:1]


def _epilogue_body(d_ref, dec_ref, keep_ref):
    d = d_ref[...]
    dec = (d > 0.0).astype(jnp.float32)
    any_active = jnp.max(dec, axis=1, keepdims=True)
    col0 = jax.lax.broadcasted_iota(jnp.int32, dec.shape, 1) == 0
    dec_ref[...] = jnp.where((any_active == 0.0) & col0, 1.0, dec)
    keep_ref[...] = 1.0 / (1.0 + jnp.exp2(d * -_LOG2E))


@jax.jit
def kernel(slots, W1, b1, W2, b2, global_step):
    B, N, DIM = slots.shape
    F = W1.shape[1]
    x = slots.reshape(B * N, DIM)
    b1r = b1.reshape(1, F)
    b2r = b2.reshape(1, 2)

    grid = (B * N // _TR,)
    d_col = pl.pallas_call(
        _mlp_body,
        grid=grid,
        in_specs=[
            pl.BlockSpec((_TR, DIM), lambda i: (i, 0)),
            pl.BlockSpec((DIM, F), lambda i: (0, 0)),
            pl.BlockSpec((1, F), lambda i: (0, 0)),
            pl.BlockSpec((F, 2), lambda i: (0, 0)),
            pl.BlockSpec((1, 2), lambda i: (0, 0)),
        ],
        out_specs=pl.BlockSpec((_TR, 1), lambda i: (i, 0)),
        out_shape=jax.ShapeDtypeStruct((B * N, 1), jnp.float32),
        compiler_params=pltpu.CompilerParams(
            dimension_semantics=("arbitrary",),
        ),
    )(x, W1, b1r, W2, b2r)

    d_packed = d_col.reshape(B, N)
    out = pl.pallas_call(
        _epilogue_body,
        out_shape=[
            jax.ShapeDtypeStruct((B, N), jnp.float32),
            jax.ShapeDtypeStruct((B, N), jnp.float32),
        ],
    )(d_packed)
    return (out[0], out[1])
